# Initial kernel scaffold; baseline (speedup 1.0000x reference)
import jax
import jax.numpy as jnp
from jax.experimental import pallas as pl


def kernel(x, pos_embed):
    raise NotImplementedError("write your pallas kernel here")


# speedup vs baseline: 3.0291x; 3.0291x over previous
import jax
import jax.numpy as jnp
from jax.experimental import pallas as pl


def _body(pe_ref, x_ref, out_ref):
    d = x_ref.shape[-1]
    out_ref[:, :, :d] = x_ref[...]
    out_ref[:, :, d:] = pe_ref[...][None]


def kernel(x, pos_embed):
    b, s, d = x.shape
    bs = 512
    grid = (s // bs, b)
    return pl.pallas_call(
        _body,
        grid=grid,
        in_specs=[
            pl.BlockSpec((bs, d), lambda i, j: (i, 0)),
            pl.BlockSpec((1, bs, d), lambda i, j: (j, i, 0)),
        ],
        out_specs=pl.BlockSpec((1, bs, 2 * d), lambda i, j: (j, i, 0)),
        out_shape=jax.ShapeDtypeStruct((b, s, 2 * d), x.dtype),
    )(pos_embed, x)
